# Initial kernel scaffold; baseline (speedup 1.0000x reference)
import jax
import jax.numpy as jnp
from jax.experimental import pallas as pl


def kernel(x, table):
    raise NotImplementedError("write your pallas kernel here")


# speedup vs baseline: 5.1000x; 5.1000x over previous
import jax
import jax.numpy as jnp
from jax import lax
from jax.experimental import pallas as pl
from jax.experimental.pallas import tpu as pltpu

_MAX = 64
_S = 512
_H = 256


def _body(x_ref, t_ref, o_ref, d8_ref):
    i = pl.program_id(0)

    @pl.when(i == 0)
    def _init():
        for p in range(8):
            d8_ref[p, 0 : 448 - p] = jnp.broadcast_to(t_ref[0:1], (448 - p, _H))
            d8_ref[p, 448 - p : 576 - p] = t_ref[1:129]
            d8_ref[p, 576 - p : 1024] = jnp.broadcast_to(
                t_ref[128:129], (448 + p, _H)
            )

    off = (_S - 1) - i
    p = lax.rem(off, 8)
    base = pl.multiple_of(off - p, 8)
    o_ref[0] = x_ref[0] + d8_ref[p, pl.ds(base, _S)]


def kernel(x, table):
    out = pl.pallas_call(
        _body,
        grid=(_S,),
        in_specs=[
            pl.BlockSpec((1, _S, _H), lambda i: (0, 0, 0)),
            pl.BlockSpec((2 * _MAX + 1, _H), lambda i: (0, 0)),
        ],
        out_specs=pl.BlockSpec((1, _S, _H), lambda i: (i, 0, 0)),
        out_shape=jax.ShapeDtypeStruct((_S, _S, _H), jnp.float32),
        scratch_shapes=[pltpu.VMEM((8, 2 * _S, _H), jnp.float32)],
    )(x, table)
    return out
